# Initial kernel scaffold; baseline (speedup 1.0000x reference)
import jax
import jax.numpy as jnp
from jax.experimental import pallas as pl


def kernel(hidden_states, attn_weights, metric, text_emb):
    raise NotImplementedError("write your pallas kernel here")


# speedup vs baseline: 1.1364x; 1.1364x over previous
import functools

import jax
import jax.numpy as jnp
from jax import lax
from jax.experimental import pallas as pl
from jax.experimental.pallas import tpu as pltpu
from jax.experimental.pallas import tpu_sc as plsc

F32 = jnp.float32
I32 = jnp.int32

B = 8
H = 16
LV = 577
PD = 576
D = 1024
CK = 64
DOM = 54
NSEL = DOM + 1
CTX = 10
NR = LV - NSEL
NM = NR - CTX
STEP = NR // CTX
LAST_T = STEP * (CTX - 1)
BIG = 3.4e38
TAB = 768
SLOT = 80
TRASH = 75


def _mesh():
    return plsc.VectorSubcoreMesh(core_axis_name="c", subcore_axis_name="s")


_PARAMS = pltpu.CompilerParams(needs_layout_passes=False)


def _rsqrt(s):
    i = plsc.bitcast(s, I32)
    y = plsc.bitcast(jnp.full((16,), 0x5F3759DF, I32) - (i >> 1), F32)
    for _ in range(3):
        y = y * (1.5 - 0.5 * s * y * y)
    return y


def _inv_norm(ss, eps):
    s = jnp.maximum(ss, 1e-37)
    return 1.0 / (s * _rsqrt(s) + eps)


def _score_body(attn, metric, text, tab,
                abuf, mv, txv, tnb, sdb, csb, scb, flg, remb, mrgb, tnorm,
                tabv, sem):
    core = lax.axis_index("c")
    sub = lax.axis_index("s")
    b = core * 4 + sub
    iota = lax.iota(I32, 16)
    lane0 = iota == 0

    @pl.when(sub < 4)
    def _():
        pltpu.sync_copy(metric.at[b], mv)
        pltpu.sync_copy(text, txv)

        for cc in range(3):
            pltpu.sync_copy(attn.at[b, pl.ds(cc * 192, 192)], abuf)

            def sd_body(j, c, _cc=cc):
                rows = iota + j * 16
                acc = jnp.zeros((16,), F32)
                for h in range(H):
                    acc = acc + plsc.load_gather(
                        abuf, [rows, jnp.full((16,), h, I32)])
                sdb[pl.ds(_cc * 192 + j * 16, 16)] = acc
                return c

            lax.fori_loop(0, 12, sd_body, 0)

        bfull = jnp.full((16,), b, I32)
        tvecs = [plsc.load_gather(txv, [bfull, iota + 16 * c])
                 for c in range(4)]
        ssq = jnp.zeros((16,), F32)
        for t in tvecs:
            ssq = ssq + t * t
        ss = jnp.sum(ssq)
        invt = _inv_norm(jnp.full((16,), ss, F32), 1e-12)
        for c in range(4):
            tnb[pl.ds(16 * c, 16)] = tvecs[c] * invt

        def cos_body(j, c):
            rows = iota + j * 16 + 1
            dot = jnp.zeros((16,), F32)
            sq = jnp.zeros((16,), F32)
            for cb in range(4):
                tnv = tnb[pl.ds(cb * 16, 16)]
                for ci in range(16):
                    col = plsc.load_gather(
                        mv, [rows, jnp.full((16,), cb * 16 + ci, I32)])
                    dot = dot + col * tnv[ci]
                    sq = sq + col * col
            csb[pl.ds(j * 16, 16)] = dot * _inv_norm(sq, 1e-12)
            return c

        lax.fori_loop(0, 36, cos_body, 0)

        def zstats(buf):
            def s1(j, acc):
                return acc + buf[pl.ds(j * 16, 16)]

            tot = lax.fori_loop(0, 36, s1, jnp.zeros((16,), F32))
            mfull = (jnp.full((16,), jnp.sum(tot), F32)
                     / jnp.full((16,), float(PD), F32))

            def s2(j, acc):
                d = buf[pl.ds(j * 16, 16)] - mfull
                return acc + d * d

            v2 = lax.fori_loop(0, 36, s2, jnp.zeros((16,), F32))
            varv = (jnp.full((16,), jnp.sum(v2), F32)
                    / jnp.full((16,), float(PD - 1), F32))
            varv = jnp.maximum(varv, 1e-37)
            std = varv * _rsqrt(varv)
            return mfull, 1.0 / (std + 1e-6)

        m1, i1 = zstats(sdb)
        m2, i2 = zstats(csb)

        def blend(j, c):
            sl = pl.ds(j * 16, 16)
            scb[sl] = 0.5 * ((sdb[sl] - m1) * i1) + 0.5 * ((csb[sl] - m2) * i2)
            return c

        lax.fori_loop(0, 36, blend, 0)

        def fz(j, c):
            flg[pl.ds(j * 16, 16)] = jnp.zeros((16,), I32)
            return c

        lax.fori_loop(0, 37, fz, 0)
        plsc.store_scatter(flg, [jnp.zeros((16,), I32)],
                           jnp.full((16,), 1, I32), mask=lane0)

        def topk_body(k, c):
            def find(j, carry):
                bv, bi = carry
                s = scb[pl.ds(j * 16, 16)]
                ii = iota + j * 16
                gt = s > bv
                return jnp.where(gt, s, bv), jnp.where(gt, ii, bi)

            bv, bi = lax.fori_loop(
                0, 36, find,
                (jnp.full((16,), -BIG, F32), jnp.zeros((16,), I32)))
            gm = jnp.max(bv)
            gi = jnp.min(jnp.where(bv == gm, bi, 1 << 20))
            gif = jnp.full((16,), gi, I32)
            plsc.store_scatter(scb, [gif], jnp.full((16,), -BIG, F32),
                               mask=lane0)
            plsc.store_scatter(flg, [gif + 1], jnp.full((16,), 1, I32),
                               mask=lane0)
            return c

        lax.fori_loop(0, DOM, topk_body, 0)

        def cpt(j, carry):
            selv, remv = carry
            f = flg[pl.ds(j * 16, 16)]
            pos = iota + j * 16
            valid = pos < LV
            ms = f == 1
            msi = ms.astype(I32)
            rank = selv + plsc.cumsum(msi) - msi
            plsc.store_scatter(tabv, [jnp.zeros((16,), I32), pos], rank,
                               mask=ms)
            mr = (f == 0) & valid
            mri = mr.astype(I32)
            rrank = remv + plsc.cumsum(mri) - mri
            plsc.store_scatter(remb, [rrank], pos, mask=mr)
            return (selv + plsc.all_reduce_population_count(ms),
                    remv + plsc.all_reduce_population_count(mr))

        lax.fori_loop(0, 37, cpt,
                      (jnp.zeros((16,), I32), jnp.zeros((16,), I32)))

        tposv = plsc.load_gather(remb, [jnp.minimum(iota * STEP, LAST_T)])
        plsc.store_scatter(tabv, [jnp.zeros((16,), I32), tposv],
                           NSEL + iota, mask=iota < CTX)

        for t in range(CTX):
            rt = jnp.full((16,), tposv[t], I32)
            tt = [plsc.load_gather(mv, [rt, iota + 16 * c]) for c in range(4)]
            tsq = jnp.zeros((16,), F32)
            for v in tt:
                tsq = tsq + v * v
            tinv = _inv_norm(jnp.full((16,), jnp.sum(tsq), F32), 1e-12)
            for c in range(4):
                tnorm[pl.ds(t * 64 + 16 * c, 16)] = tt[c] * tinv

        def mcpt(j, carry):
            pos_r = iota + j * 16
            valid = pos_r < NR
            ist = ((pos_r % STEP) == 0) & (pos_r <= LAST_T)
            mm = valid & (~ist)
            mmi = mm.astype(I32)
            rk = carry + plsc.cumsum(mmi) - mmi
            v = remb[pl.ds(j * 16, 16)]
            plsc.store_scatter(mrgb, [rk], v, mask=mm)
            return carry + plsc.all_reduce_population_count(mm)

        lax.fori_loop(0, 33, mcpt, jnp.zeros((16,), I32))

        def asn(j, cnt):
            mpos = mrgb[pl.ds(j * 16, 16)]
            accs = [jnp.zeros((16,), F32) for _ in range(CTX)]
            for cb in range(4):
                tnvs = [tnorm[pl.ds(t * 64 + cb * 16, 16)]
                        for t in range(CTX)]
                for ci in range(16):
                    col = plsc.load_gather(
                        mv, [mpos, jnp.full((16,), cb * 16 + ci, I32)])
                    for t in range(CTX):
                        accs[t] = accs[t] + col * tnvs[t][ci]
            bestv = accs[0]
            besti = jnp.zeros((16,), I32)
            for t in range(1, CTX):
                gt = accs[t] > bestv
                bestv = jnp.where(gt, accs[t], bestv)
                besti = jnp.where(gt, jnp.full((16,), t, I32), besti)
            plsc.store_scatter(tabv, [jnp.zeros((16,), I32), mpos],
                               (NSEL + CTX) + besti)
            for t in range(CTX):
                pc = plsc.all_reduce_population_count(besti == t)
                cnt = cnt + jnp.where(iota == t, pc.astype(F32), 0.0)
            return cnt

        cnt = lax.fori_loop(0, NM // 16, asn, jnp.zeros((16,), F32))
        invc = 1.0 / jnp.maximum(cnt, 1.0)
        tabv[0, pl.ds(640, 16)] = plsc.bitcast(invc, I32)

        pltpu.sync_copy(tabv, tab.at[b])


def _merge_body(hidden, tab, out, tabv, chunk, zb, fbuf, sidx, sidx1,
                shared, sem):
    core = lax.axis_index("c")
    sub = lax.axis_index("s")
    b = core * 4 + sub // 4
    s = sub % 4
    lb = sub // 4
    base = lb * SLOT
    iota = lax.iota(I32, 16)
    basev = jnp.full((16,), base, I32)

    pltpu.sync_copy(tab.at[b], tabv)

    for r in range(8):
        for c in range(8):
            zb[r, c, pl.ds(0, 16)] = jnp.zeros((16,), F32)
            zb[r, c, pl.ds(16, 16)] = jnp.zeros((16,), F32)
            zb[r, c, pl.ds(32, 16)] = jnp.zeros((16,), F32)
            zb[r, c, pl.ds(48, 16)] = jnp.zeros((16,), F32)
            zb[r, c, pl.ds(64, 16)] = jnp.zeros((16,), F32)
            zb[r, c, pl.ds(80, 16)] = jnp.zeros((16,), F32)
            zb[r, c, pl.ds(96, 16)] = jnp.zeros((16,), F32)
            zb[r, c, pl.ds(112, 16)] = jnp.zeros((16,), F32)

    @pl.when(s == 0)
    def _():
        for q in range(SLOT // 8):
            pltpu.sync_copy(zb, shared.at[pl.ds(base + q * 8, 8)])

    plsc.subcore_barrier()

    def do_chunk(off):
        for u in range(2):
            sidx[pl.ds(u * 16, 16)] = tabv[0, pl.ds(off + u * 16, 16)] + basev
        pltpu.sync_copy(hidden.at[b, pl.ds(off, 32)], chunk)
        pltpu.sync_copy(chunk, shared.at[sidx], add=True)

    for u in range(4):
        do_chunk((s + 4 * u) * 32)

    @pl.when(s == 0)
    def _():
        do_chunk(512)

    @pl.when(s == 1)
    def _():
        do_chunk(544)

    @pl.when(s == 3)
    def _():
        d576 = tabv[0, pl.ds(561, 16)][15]
        plsc.store_scatter(sidx1, [jnp.zeros((16,), I32)],
                           jnp.full((16,), d576, I32) + basev,
                           mask=iota == 0)
        pltpu.sync_copy(hidden.at[b, pl.ds(576, 1)], chunk.at[pl.ds(0, 1)])
        pltpu.sync_copy(chunk.at[pl.ds(0, 1)], shared.at[sidx1], add=True)

    plsc.subcore_barrier()

    @pl.when(s == 0)
    def _():
        pltpu.sync_copy(shared.at[pl.ds(base, 48)], out.at[b, pl.ds(0, 48)])
        pltpu.sync_copy(shared.at[pl.ds(base + 48, 27)], fbuf)
        invv = plsc.bitcast(tabv[0, pl.ds(640, 16)], F32)
        for t in range(CTX):
            iv = invv[t]
            for r in range(8):
                for cs in range(8):
                    sl = pl.ds(cs * 16, 16)
                    fbuf[7 + t, r, sl] = (fbuf[7 + t, r, sl]
                                          + fbuf[17 + t, r, sl] * iv)
        pltpu.sync_copy(fbuf.at[pl.ds(0, 17)], out.at[b, pl.ds(48, 17)])


@functools.partial(
    pl.kernel,
    out_type=jax.ShapeDtypeStruct((B, 1, TAB), I32),
    mesh=_mesh(),
    compiler_params=_PARAMS,
    scratch_types=[
        pltpu.VMEM((192, 16), F32),
        pltpu.VMEM((LV, CK), F32),
        pltpu.VMEM((B, CK), F32),
        pltpu.VMEM((CK,), F32),
        pltpu.VMEM((PD,), F32),
        pltpu.VMEM((PD,), F32),
        pltpu.VMEM((PD,), F32),
        pltpu.VMEM((592,), I32),
        pltpu.VMEM((528,), I32),
        pltpu.VMEM((NM,), I32),
        pltpu.VMEM((CTX * CK,), F32),
        pltpu.VMEM((1, TAB), I32),
        pltpu.SemaphoreType.DMA,
    ],
)
def _score_kernel(attn, metric, text, tab, *rest):
    _score_body(attn, metric, text, tab, *rest)


@functools.partial(
    pl.kernel,
    out_type=jax.ShapeDtypeStruct((B, NSEL + CTX, 8, 128), F32),
    mesh=_mesh(),
    compiler_params=_PARAMS,
    scratch_types=[
        pltpu.VMEM((1, TAB), I32),
        pltpu.VMEM((32, 8, 128), F32),
        pltpu.VMEM((8, 8, 128), F32),
        pltpu.VMEM((27, 8, 128), F32),
        pltpu.VMEM((32,), I32),
        pltpu.VMEM((1,), I32),
        pltpu.VMEM_SHARED((4 * SLOT, 8, 128), F32),
        pltpu.SemaphoreType.DMA,
    ],
)
def _merge_kernel(hidden, tab, out, *rest):
    _merge_body(hidden, tab, out, *rest)


def kernel(hidden_states, attn_weights, metric, text_emb):
    acls = jnp.swapaxes(attn_weights[:, :, 0, 1:], 1, 2)
    tab = _score_kernel(acls, metric, text_emb)
    hidden4 = hidden_states.reshape(B, LV, 8, 128)
    out4 = _merge_kernel(hidden4, tab)
    return out4.reshape(B, NSEL + CTX, D)


# speedup vs baseline: 1.6201x; 1.4256x over previous
import functools

import jax
import jax.numpy as jnp
from jax import lax
from jax.experimental import pallas as pl
from jax.experimental.pallas import tpu as pltpu
from jax.experimental.pallas import tpu_sc as plsc

F32 = jnp.float32
I32 = jnp.int32

B = 8
H = 16
LV = 577
PD = 576
D = 1024
CK = 64
DOM = 54
NSEL = DOM + 1
CTX = 10
NR = LV - NSEL
NM = NR - CTX
STEP = NR // CTX
LAST_T = STEP * (CTX - 1)
BIG = 3.4e38
TAB = 768
RT = 8
QP = PD // 4
SLOT = 80
TRASH = 75
FBASE = 4096
IBASE = 2048


def _mesh():
    return plsc.VectorSubcoreMesh(core_axis_name="c", subcore_axis_name="s")


_PARAMS = pltpu.CompilerParams(needs_layout_passes=False)


def _rsqrt(s):
    i = plsc.bitcast(s, I32)
    y = plsc.bitcast(jnp.full((16,), 0x5F3759DF, I32) - (i >> 1), F32)
    for _ in range(3):
        y = y * (1.5 - 0.5 * s * y * y)
    return y


def _inv_norm(ss, eps):
    s = jnp.maximum(ss, 1e-37)
    return 1.0 / (s * _rsqrt(s) + eps)


def _score_body(attn, metric, text, tab, abuf, mv, txv, tnb, sdp, csp, sdb,
                csb, scb, bmx, flg, remb, mrgb, tnormb, tabrow, commf32,
                commi32, sem, sem2):
    core = lax.axis_index("c")
    sub = lax.axis_index("s")
    b = core * 4 + sub // 4
    u = sub % 4
    lb = sub // 4
    fb = lb * FBASE
    ib = lb * IBASE
    iota = lax.iota(I32, 16)
    lane0 = iota == 0

    c1 = pltpu.async_copy(metric.at[b], mv, sem)
    c2 = pltpu.async_copy(attn.at[b, pl.ds(u * QP, QP)], abuf, sem2)
    c3 = pltpu.async_copy(text, txv, sem)
    c2.wait()

    def sd_body(j, c):
        rows = iota + j * 16
        acc = jnp.zeros((16,), F32)
        for h in range(H):
            acc = acc + plsc.load_gather(abuf, [rows, jnp.full((16,), h, I32)])
        sdp[pl.ds(j * 16, 16)] = acc
        return c

    lax.fori_loop(0, QP // 16, sd_body, 0)
    c1.wait()
    c3.wait()

    bfull = jnp.full((16,), b, I32)
    tvecs = [plsc.load_gather(txv, [bfull, iota + 16 * c]) for c in range(4)]
    ssq = jnp.zeros((16,), F32)
    for t in tvecs:
        ssq = ssq + t * t
    invt = _inv_norm(jnp.full((16,), jnp.sum(ssq), F32), 1e-12)
    for c in range(4):
        tnb[pl.ds(16 * c, 16)] = tvecs[c] * invt

    def cos_body(j, c):
        rows = iota + j * 16 + u * QP + 1
        dot = jnp.zeros((16,), F32)
        sq = jnp.zeros((16,), F32)
        for cb in range(4):
            tnv = tnb[pl.ds(cb * 16, 16)]
            for ci in range(16):
                col = plsc.load_gather(
                    mv, [rows, jnp.full((16,), cb * 16 + ci, I32)])
                dot = dot + col * tnv[ci]
                sq = sq + col * col
        csp[pl.ds(j * 16, 16)] = dot * _inv_norm(sq, 1e-12)
        return c

    lax.fori_loop(0, QP // 16, cos_body, 0)

    pltpu.sync_copy(sdp, commf32.at[pl.ds(fb + u * 256, QP)])
    pltpu.sync_copy(csp, commf32.at[pl.ds(fb + 1024 + u * 256, QP)])

    plsc.subcore_barrier()

    @pl.when(u == 0)
    def _():
        for w in range(4):
            pltpu.sync_copy(commf32.at[pl.ds(fb + w * 256, QP)],
                            sdb.at[pl.ds(w * QP, QP)])
            pltpu.sync_copy(commf32.at[pl.ds(fb + 1024 + w * 256, QP)],
                            csb.at[pl.ds(w * QP, QP)])

        def zstats(buf):
            def s1(j, acc):
                return acc + buf[pl.ds(j * 16, 16)]

            tot = lax.fori_loop(0, 36, s1, jnp.zeros((16,), F32))
            mfull = (jnp.full((16,), jnp.sum(tot), F32)
                     / jnp.full((16,), float(PD), F32))

            def s2(j, acc):
                dd = buf[pl.ds(j * 16, 16)] - mfull
                return acc + dd * dd

            v2 = lax.fori_loop(0, 36, s2, jnp.zeros((16,), F32))
            varv = (jnp.full((16,), jnp.sum(v2), F32)
                    / jnp.full((16,), float(PD - 1), F32))
            varv = jnp.maximum(varv, 1e-37)
            std = varv * _rsqrt(varv)
            return mfull, 1.0 / (std + 1e-6)

        m1, i1 = zstats(sdb)
        m2, i2 = zstats(csb)

        def blend(j, c):
            sl = pl.ds(j * 16, 16)
            scb[sl] = (0.5 * ((sdb[sl] - m1) * i1)
                       + 0.5 * ((csb[sl] - m2) * i2))
            return c

        lax.fori_loop(0, 36, blend, 0)


        def bm(j, c):
            mxv = jnp.max(scb[pl.ds(j * 16, 16)])
            plsc.store_scatter(bmx, [jnp.full((16,), j, I32)],
                               jnp.full((16,), mxv, F32), mask=lane0)
            return c

        lax.fori_loop(0, 36, bm, 0)
        bmx[pl.ds(32, 16)] = jnp.where(
            iota < 4, bmx[pl.ds(32, 16)], jnp.full((16,), -BIG, F32))

        def fz(j, c):
            flg[pl.ds(j * 16, 16)] = jnp.zeros((16,), I32)
            return c

        lax.fori_loop(0, 37, fz, 0)
        plsc.store_scatter(flg, [jnp.zeros((16,), I32)],
                           jnp.full((16,), 1, I32), mask=lane0)

        def topk_body(k, c):
            bv = jnp.full((16,), -BIG, F32)
            bi = jnp.zeros((16,), I32)
            for q in range(3):
                sq_ = bmx[pl.ds(q * 16, 16)]
                ii = iota + q * 16
                gt = sq_ > bv
                bv = jnp.where(gt, sq_, bv)
                bi = jnp.where(gt, ii, bi)
            gm = jnp.max(bv)
            jb = jnp.min(jnp.where(bv == gm, bi, 1 << 20))
            s = scb[pl.ds(jb * 16, 16)]
            li = jnp.min(jnp.where(s == gm, iota, 1 << 20))
            gi = jb * 16 + li
            snew = jnp.where(iota == li, jnp.full((16,), -BIG, F32), s)
            scb[pl.ds(jb * 16, 16)] = snew
            plsc.store_scatter(bmx, [jnp.full((16,), jb, I32)],
                               jnp.full((16,), jnp.max(snew), F32),
                               mask=lane0)
            plsc.store_scatter(flg, [jnp.full((16,), gi + 1, I32)],
                               jnp.full((16,), 1, I32), mask=lane0)
            return c

        lax.fori_loop(0, DOM, topk_body, 0)

        def cpt(j, carry):
            selv, remv = carry
            f = flg[pl.ds(j * 16, 16)]
            pos = iota + j * 16
            valid = pos < LV
            ms = f == 1
            msi = ms.astype(I32)
            rank = selv + plsc.cumsum(msi) - msi
            plsc.store_scatter(tabrow, [pos], rank, mask=ms)
            mr = (f == 0) & valid
            mri = mr.astype(I32)
            rrank = remv + plsc.cumsum(mri) - mri
            plsc.store_scatter(remb, [rrank], pos, mask=mr)
            return (selv + plsc.all_reduce_population_count(ms),
                    remv + plsc.all_reduce_population_count(mr))

        lax.fori_loop(0, 37, cpt,
                      (jnp.zeros((16,), I32), jnp.zeros((16,), I32)))

        tposv = plsc.load_gather(remb, [jnp.minimum(iota * STEP, LAST_T)])
        plsc.store_scatter(tabrow, [tposv], NSEL + iota, mask=iota < CTX)

        for t in range(CTX):
            rt = jnp.full((16,), tposv[t], I32)
            tt = [plsc.load_gather(mv, [rt, iota + 16 * c]) for c in range(4)]
            tsq = jnp.zeros((16,), F32)
            for v in tt:
                tsq = tsq + v * v
            tinv = _inv_norm(jnp.full((16,), jnp.sum(tsq), F32), 1e-12)
            for c in range(4):
                tnormb[pl.ds(t * 64 + 16 * c, 16)] = tt[c] * tinv

        def mcpt(j, carry):
            pos_r = iota + j * 16
            valid = pos_r < NR
            ist = ((pos_r % STEP) == 0) & (pos_r <= LAST_T)
            mm = valid & (~ist)
            mmi = mm.astype(I32)
            rk = carry + plsc.cumsum(mmi) - mmi
            v = remb[pl.ds(j * 16, 16)]
            plsc.store_scatter(mrgb, [rk], v, mask=mm)
            return carry + plsc.all_reduce_population_count(mm)

        lax.fori_loop(0, 33, mcpt, jnp.zeros((16,), I32))

        pltpu.sync_copy(tnormb, commf32.at[pl.ds(fb + 2048, CTX * CK)])
        pltpu.sync_copy(mrgb, commi32.at[pl.ds(ib, NM)])
        tb = b * (RT * TAB)
        pltpu.sync_copy(tabrow, tab.at[pl.ds(tb, TAB)])
        pltpu.sync_copy(mrgb, tab.at[pl.ds(tb + TAB, NM)])

    plsc.subcore_barrier()

    pltpu.sync_copy(commi32.at[pl.ds(ib + u * 128, 128)],
                    mrgb.at[pl.ds(0, 128)])
    pltpu.sync_copy(commf32.at[pl.ds(fb + 2048, CTX * CK)], tnormb)

    def asn(j, cnt):
        mpos = mrgb[pl.ds(j * 16, 16)]
        accs = [jnp.zeros((16,), F32) for _ in range(CTX)]
        for cb in range(4):
            tnvs = [tnormb[pl.ds(t * 64 + cb * 16, 16)] for t in range(CTX)]
            for ci in range(16):
                col = plsc.load_gather(
                    mv, [mpos, jnp.full((16,), cb * 16 + ci, I32)])
                for t in range(CTX):
                    accs[t] = accs[t] + col * tnvs[t][ci]
        bestv = accs[0]
        besti = jnp.zeros((16,), I32)
        for t in range(1, CTX):
            gt = accs[t] > bestv
            bestv = jnp.where(gt, accs[t], bestv)
            besti = jnp.where(gt, jnp.full((16,), t, I32), besti)
        tabrow[pl.ds(j * 16, 16)] = besti
        for t in range(CTX):
            pc = plsc.all_reduce_population_count(besti == t)
            cnt = cnt + jnp.where(iota == t, pc.astype(F32), 0.0)
        return cnt

    cnt = lax.fori_loop(0, 8, asn, jnp.zeros((16,), F32))
    tabrow[pl.ds(640, 16)] = plsc.bitcast(cnt, I32)
    tb2 = b * (RT * TAB) + (2 + u) * TAB
    pltpu.sync_copy(tabrow.at[pl.ds(0, 128)], tab.at[pl.ds(tb2, 128)])
    pltpu.sync_copy(tabrow.at[pl.ds(640, 16)], tab.at[pl.ds(tb2 + 640, 16)])


def _merge_body(hidden, tab, out, tabv, cha, chb, sidxa, sidxb, sidx1,
                shared, sem):
    core = lax.axis_index("c")
    sub = lax.axis_index("s")
    b = core * 4 + sub // 4
    s = sub % 4
    lb = sub // 4
    base = lb * SLOT
    iota = lax.iota(I32, 16)
    basev = jnp.full((16,), base, I32)

    pltpu.sync_copy(tab.at[pl.ds(b * (RT * TAB), 6 * TAB)], tabv)

    for w in range(4):
        def rec(j, c, _w=w):
            apos = tabv[pl.ds((2 + _w) * TAB + j * 16, 16)]
            mpos = tabv[pl.ds(TAB + _w * 128 + j * 16, 16)]
            plsc.store_scatter(tabv, [mpos], (NSEL + CTX) + apos)
            return c

        lax.fori_loop(0, 8, rec, 0)

    for r in range(16):
        for c8 in range(8):
            for cs in range(8):
                cha[r, c8, pl.ds(cs * 16, 16)] = jnp.zeros((16,), F32)

    @pl.when(s == 0)
    def _():
        for q in range(SLOT // 16):
            pltpu.sync_copy(cha.at[pl.ds(0, 16)],
                            shared.at[pl.ds(base + q * 16, 16)])

    plsc.subcore_barrier()

    def fill_sidx(sidx, off):
        sidx[pl.ds(0, 16)] = tabv[pl.ds(off, 16)] + basev
        sidx[pl.ds(16, 16)] = tabv[pl.ds(off + 16, 16)] + basev

    offs = [(s + 4 * k) * 32 for k in range(4)]
    bufs = [(cha, sidxa), (chb, sidxb)]
    cps = [None, None]
    cps[0] = pltpu.async_copy(hidden.at[b, pl.ds(offs[0], 32)], cha, sem)
    for k in range(4):
        buf, sidx = bufs[k % 2]
        if k < 3:
            nbuf, _ = bufs[(k + 1) % 2]
            cps[(k + 1) % 2] = pltpu.async_copy(
                hidden.at[b, pl.ds(offs[k + 1], 32)], nbuf, sem)
        fill_sidx(sidx, offs[k])
        cps[k % 2].wait()
        pltpu.sync_copy(buf, shared.at[sidx], add=True)

    def do_chunk_sync(off):
        fill_sidx(sidxa, off)
        pltpu.sync_copy(hidden.at[b, pl.ds(off, 32)], cha)
        pltpu.sync_copy(cha, shared.at[sidxa], add=True)

    @pl.when(s == 0)
    def _():
        do_chunk_sync(512)

    @pl.when(s == 1)
    def _():
        do_chunk_sync(544)

    @pl.when(s == 3)
    def _():
        d576 = tabv[pl.ds(561, 16)][15]
        plsc.store_scatter(sidx1, [jnp.zeros((16,), I32)],
                           jnp.full((16,), d576, I32) + basev,
                           mask=iota == 0)
        pltpu.sync_copy(hidden.at[b, pl.ds(576, 1)], cha.at[pl.ds(0, 1)])
        pltpu.sync_copy(cha.at[pl.ds(0, 1)], shared.at[sidx1], add=True)

    plsc.subcore_barrier()

    @pl.when(s == 0)
    def _():
        pltpu.sync_copy(shared.at[pl.ds(base, 48)], out.at[b, pl.ds(0, 48)])
        pltpu.sync_copy(shared.at[pl.ds(base + 48, 17)], cha.at[pl.ds(0, 17)])
        pltpu.sync_copy(shared.at[pl.ds(base + 65, CTX)],
                        chb.at[pl.ds(0, CTX)])
        cnt = jnp.zeros((16,), F32)
        for w in range(4):
            cnt = cnt + plsc.bitcast(tabv[pl.ds((2 + w) * TAB + 640, 16)], F32)
        invv = 1.0 / jnp.maximum(cnt, 1.0)
        for t in range(CTX):
            iv = invv[t]
            for r in range(8):
                for cs in range(8):
                    sl = pl.ds(cs * 16, 16)
                    cha[7 + t, r, sl] = (cha[7 + t, r, sl]
                                         + chb[t, r, sl] * iv)
        pltpu.sync_copy(cha.at[pl.ds(0, 17)], out.at[b, pl.ds(48, 17)])


@functools.partial(
    pl.kernel,
    out_type=jax.ShapeDtypeStruct((B * RT * TAB,), I32),
    mesh=_mesh(),
    compiler_params=_PARAMS,
    scratch_types=[
        pltpu.VMEM((QP, H), F32),
        pltpu.VMEM((LV, CK), F32),
        pltpu.VMEM((B, CK), F32),
        pltpu.VMEM((CK,), F32),
        pltpu.VMEM((QP,), F32),
        pltpu.VMEM((QP,), F32),
        pltpu.VMEM((PD,), F32),
        pltpu.VMEM((PD,), F32),
        pltpu.VMEM((PD,), F32),
        pltpu.VMEM((48,), F32),
        pltpu.VMEM((592,), I32),
        pltpu.VMEM((528,), I32),
        pltpu.VMEM((NM,), I32),
        pltpu.VMEM((CTX * CK,), F32),
        pltpu.VMEM((TAB,), I32),
        pltpu.VMEM_SHARED((4 * FBASE,), F32),
        pltpu.VMEM_SHARED((4 * IBASE,), I32),
        pltpu.SemaphoreType.DMA,
        pltpu.SemaphoreType.DMA,
    ],
)
def _score_kernel(attn, metric, text, tab, *rest):
    _score_body(attn, metric, text, tab, *rest)


@functools.partial(
    pl.kernel,
    out_type=jax.ShapeDtypeStruct((B, NSEL + CTX, 8, 128), F32),
    mesh=_mesh(),
    compiler_params=_PARAMS,
    scratch_types=[
        pltpu.VMEM((6 * TAB,), I32),
        pltpu.VMEM((32, 8, 128), F32),
        pltpu.VMEM((32, 8, 128), F32),
        pltpu.VMEM((32,), I32),
        pltpu.VMEM((32,), I32),
        pltpu.VMEM((1,), I32),
        pltpu.VMEM_SHARED((4 * SLOT, 8, 128), F32),
        pltpu.SemaphoreType.DMA,
    ],
)
def _merge_kernel(hidden, tab, out, *rest):
    _merge_body(hidden, tab, out, *rest)


def kernel(hidden_states, attn_weights, metric, text_emb):
    acls = jnp.swapaxes(attn_weights[:, :, 0, 1:], 1, 2)
    tab = _score_kernel(acls, metric, text_emb)
    hidden4 = hidden_states.reshape(B, LV, 8, 128)
    out4 = _merge_kernel(hidden4, tab)
    return out4.reshape(B, NSEL + CTX, D)
